# baseline (speedup 1.0000x reference)
import math
from functools import partial

import jax
import jax.numpy as jnp
from jax.experimental import pallas as pl
from jax.experimental.pallas import tpu as pltpu

_MIB = 1024 * 1024

_STAGES = (
    (1, 3, 1, 32, 16, 2),
    (6, 3, 2, 16, 24, 3),
    (6, 5, 2, 24, 48, 3),
    (6, 3, 2, 48, 88, 4),
    (6, 5, 1, 88, 120, 4),
    (6, 5, 2, 120, 208, 5),
    (6, 3, 1, 208, 352, 2),
)

_PKEYS = ("ex_w", "ex_b", "dw_w", "dw_b", "se_w1", "se_b1",
          "se_w2", "se_b2", "pj_w", "pj_b")
_PKEYS_NOEX = _PKEYS[2:]


def _silu(v):
    return v * jax.nn.sigmoid(v)


def _const_spec(shape):
    n = len(shape)
    return pl.BlockSpec(shape, lambda b: (0,) * n)


def _pad_hw(v, pad):
    if pad == 0:
        return v
    H, W, C = v.shape
    zc = jnp.zeros((H, pad, C), v.dtype)
    v = jnp.concatenate([zc, v, zc], axis=1)
    zr = jnp.zeros((pad, W + 2 * pad, C), v.dtype)
    return jnp.concatenate([zr, v, zr], axis=0)


def _se_gate(pool, w1, b1, w2, b2):
    h = jnp.dot(pool, w1[...], preferred_element_type=jnp.float32) + b1[...]
    h = _silu(h)
    g = jnp.dot(h, w2[...], preferred_element_type=jnp.float32) + b2[...]
    return jax.nn.sigmoid(g)


def _halo_mask(hq, Hp, Wp, pad, H, W):
    ri = jax.lax.broadcasted_iota(jnp.int32, (Hp, Wp, 1), 0)
    ci = jax.lax.broadcasted_iota(jnp.int32, (Hp, Wp, 1), 1)
    inside = ((ri >= pad) & (ri < pad + H) & (ci >= pad) & (ci < pad + W))
    return jnp.where(inside, hq, 0.0)


def _project(y, acc_skip, blk, Ho, Wo, Cexp):
    (w1, b1, w2, b2, wp_ref, bp_ref) = blk
    pool = jnp.mean(y.reshape(Ho * Wo, Cexp), axis=0, keepdims=True)
    g = _se_gate(pool, w1, b1, w2, b2)
    yb = y.astype(jnp.bfloat16).reshape(Ho * Wo, Cexp) * g.astype(jnp.bfloat16)
    out = jnp.dot(yb, wp_ref[...], preferred_element_type=jnp.float32)
    out = out + bp_ref[...]
    if acc_skip is not None:
        out = out + acc_skip.astype(jnp.float32)
    Cout = out.shape[1]
    return out.astype(jnp.bfloat16).reshape(Ho, Wo, Cout)


def _run_s1_block(xpv, blk, k, expand, skip):
    pad = (k - 1) // 2
    Hp, Wp, Cin = xpv.shape
    H, W = Hp - 2 * pad, Wp - 2 * pad
    if expand:
        we_ref, be_ref = blk[0], blk[1]
        blk = blk[2:]
        Cexp = we_ref.shape[1]
        hq = jnp.dot(xpv.reshape(Hp * Wp, Cin), we_ref[...],
                     preferred_element_type=jnp.float32) + be_ref[...]
        hq = _silu(hq).reshape(Hp, Wp, Cexp)
        hb = _halo_mask(hq, Hp, Wp, pad, H, W).astype(jnp.bfloat16)
    else:
        hb = xpv
        Cexp = Cin
    wd_ref, bd_ref = blk[0], blk[1]
    acc = None
    for di in range(k):
        band = hb[di:di + H, :, :].astype(jnp.float32)
        for dj in range(k):
            t = di * k + dj
            wt = wd_ref[t:t + 1, :][None]
            c = band[:, dj:dj + W, :] * wt
            acc = c if acc is None else acc + c
    y = _silu(acc + bd_ref[...][None])
    sk = None
    if skip:
        sk = xpv[pad:pad + H, pad:pad + W, :].reshape(H * W, Cin)
    return _project(y, sk, blk[2:], H, W, Cexp)


def _run_s2_block(planes_in, blk, k, H, W):
    pad = (k - 1) // 2
    Hp, Wp = H + 2 * pad, W + 2 * pad
    Ho = (Hp - k) // 2 + 1
    Wo = (Wp - k) // 2 + 1
    we_ref, be_ref, wd_ref, bd_ref = blk[0], blk[1], blk[2], blk[3]
    Cexp = we_ref.shape[1]
    planes = []
    for r in range(2):
        for s in range(2):
            pv = planes_in[2 * r + s]
            hs, ws, Cin = pv.shape
            hq = jnp.dot(pv.reshape(hs * ws, Cin), we_ref[...],
                         preferred_element_type=jnp.float32) + be_ref[...]
            hq = _silu(hq).reshape(hs, ws, Cexp)
            ri = jax.lax.broadcasted_iota(jnp.int32, (hs, ws, 1), 0)
            ci = jax.lax.broadcasted_iota(jnp.int32, (hs, ws, 1), 1)
            inside = ((r + 2 * ri >= pad) & (r + 2 * ri < pad + H)
                      & (s + 2 * ci >= pad) & (s + 2 * ci < pad + W))
            hq = jnp.where(inside, hq, 0.0)
            planes.append(hq.astype(jnp.bfloat16))
    acc = None
    for r in range(2):
        for s in range(2):
            xr = planes[2 * r + s]
            na = (k - 1 - r) // 2 + 1
            nb = (k - 1 - s) // 2 + 1
            for a in range(na):
                band = xr[a:a + Ho, :, :].astype(jnp.float32)
                for b2_ in range(nb):
                    di, dj = 2 * a + r, 2 * b2_ + s
                    t = di * k + dj
                    wt = wd_ref[t:t + 1, :][None]
                    c = band[:, b2_:b2_ + Wo, :] * wt
                    acc = c if acc is None else acc + c
    y = _silu(acc + bd_ref[...][None])
    return _project(y, None, blk[4:], Ho, Wo, Cexp)


def _chain_body(*refs, cfgs, from_planes, s2_hw, out_pad, has_head, stem_hw, G):
    for g in range(G):
        i = 0
        planes = None
        xv = None
        if stem_hw is not None:
            H, W = stem_hw
            Hp, Wp = H + 2, W + 2
            pt_ref, ws_ref, bs_ref = refs[0], refs[1], refs[2]
            i = 3
            Cstem = ws_ref.shape[1]
            h = jnp.dot(pt_ref[g].reshape(Hp * Wp, 27), ws_ref[...],
                        preferred_element_type=jnp.float32) + bs_ref[...]
            h = _silu(h).reshape(Hp, Wp, Cstem)
            xv = _halo_mask(h, Hp, Wp, 1, H, W).astype(jnp.bfloat16)
        elif from_planes:
            v = refs[0][g]
            Cin = v.shape[3] // 2
            planes = [v[:, r, :, s * Cin:(s + 1) * Cin]
                      for r in range(2) for s in range(2)]
            i = 1
        else:
            xv = refs[0][g]
            i = 1

        out = None
        for bi, (k, expand, skip, is_s2) in enumerate(cfgs):
            n = 10 if expand else 8
            blk = refs[i:i + n]
            i += n
            if is_s2:
                out = _run_s2_block(planes, blk, k, s2_hw[0], s2_hw[1])
            else:
                out = _run_s1_block(xv, blk, k, expand, skip)
            if bi + 1 < len(cfgs):
                xv = _pad_hw(out, (cfgs[bi + 1][0] - 1) // 2)

        if has_head:
            wh_ref, bh_ref, wf_ref, bf_ref = refs[i:i + 4]
            lo_ref, fe_ref = refs[i + 4], refs[i + 5]
            Ho, Wo, C = out.shape
            hh = jnp.dot(out.reshape(Ho * Wo, C), wh_ref[...],
                         preferred_element_type=jnp.float32) + bh_ref[...]
            hh = _silu(hh)
            hb = hh.astype(jnp.bfloat16).astype(jnp.float32)
            feat = jnp.mean(hb, axis=0, keepdims=True)
            lo = jnp.dot(feat, wf_ref[...], preferred_element_type=jnp.float32)
            lo_ref[g] = lo + bf_ref[...]
            fe_ref[g] = feat
        else:
            refs[i][g] = _pad_hw(out, out_pad)


def _params(step_bytes):
    return pltpu.CompilerParams(
        dimension_semantics=("parallel",),
        vmem_limit_bytes=int(min(48 * _MIB, max(32 * _MIB, 3 * int(step_bytes)))))


def _blk_refs(prm, expand):
    keys = _PKEYS if expand else _PKEYS_NOEX
    return [prm[key] for key in keys]


def _chain_call(inputs, in_specs, cfgs, prms, *, from_planes=False, s2_hw=None,
                out_shape=None, out_pad=0, head=None, stem_hw=None, flops=0,
                G=1):
    args = list(inputs)
    specs = list(in_specs)
    for cfg, prm in zip(cfgs, prms):
        for a in _blk_refs(prm, cfg[1]):
            args.append(a)
            specs.append(_const_spec(a.shape))
    B = inputs[0].shape[0]
    if head is not None:
        head_w, head_b, fc_w, fc_b = head
        for a in (head_w, head_b, fc_w, fc_b):
            args.append(a)
            specs.append(_const_spec(a.shape))
        N, ncls = head_w.shape[1], fc_w.shape[1]
        out_specs = [pl.BlockSpec((G, 1, ncls), lambda b: (b, 0, 0)),
                     pl.BlockSpec((G, 1, N), lambda b: (b, 0, 0))]
        out_sds = (jax.ShapeDtypeStruct((B, 1, ncls), jnp.float32),
                   jax.ShapeDtypeStruct((B, 1, N), jnp.float32))
    else:
        out_specs = pl.BlockSpec((G,) + out_shape[1:],
                                 lambda b: (b, 0, 0, 0))
        out_sds = jax.ShapeDtypeStruct(out_shape, jnp.bfloat16)
    wbytes = sum(int(a.size) * a.dtype.itemsize for a in args[len(inputs):])
    in_bytes = sum(int(a.size) * a.dtype.itemsize for a in inputs) // B
    step = 4 * G * in_bytes + wbytes + 8 * _MIB
    return pl.pallas_call(
        partial(_chain_body, cfgs=tuple(cfgs), from_planes=from_planes,
                s2_hw=s2_hw, out_pad=out_pad, has_head=head is not None,
                stem_hw=stem_hw, G=G),
        grid=(B // G,),
        in_specs=specs,
        out_specs=out_specs,
        out_shape=out_sds,
        compiler_params=_params(step),
        cost_estimate=pl.CostEstimate(
            flops=int(flops), transcendentals=int(flops // 16),
            bytes_accessed=int(B * in_bytes + wbytes)),
    )(*args)


def _parity_pack(x):
    B, Hp, Wp, C = x.shape
    return x.reshape(B, Hp // 2, 2, Wp // 2, 2 * C)


def _chain_flops(cfgs, prms, hw_in):
    f = 0
    h = hw_in
    for (k, expand, _skip, is_s2), prm in zip(cfgs, prms):
        cexp = prm["dw_w"].shape[1]
        cout = prm["pj_w"].shape[1]
        ho = h // 2 if is_s2 else h
        cin = prm["ex_w"].shape[0] if expand else cexp
        f += 2 * (h * h * cin * cexp * int(expand)
                  + k * k * ho * ho * cexp + ho * ho * cexp * cout)
        h = ho
    return f


def _unpack(ws):
    fc_b, fc_w, head_b, head_w = ws[0], ws[1], ws[2], ws[3]
    i = 4
    stages = []
    for (expand, _k, _s, _ci, _co, layers) in _STAGES:
        blocks = []
        for _li in range(layers):
            if expand == 1:
                dw_b, dw_w, pj_b, pj_w, b1, b2, w1, w2 = ws[i:i + 8]
                i += 8
                blk = dict(dw_w=dw_w, dw_b=dw_b, pj_w=pj_w, pj_b=pj_b,
                           se_w1=w1, se_b1=b1, se_w2=w2, se_b2=b2)
            else:
                dw_b, dw_w, ex_b, ex_w, pj_b, pj_w, b1, b2, w1, w2 = ws[i:i + 10]
                i += 10
                blk = dict(dw_w=dw_w, dw_b=dw_b, ex_w=ex_w, ex_b=ex_b,
                           pj_w=pj_w, pj_b=pj_b,
                           se_w1=w1, se_b1=b1, se_w2=w2, se_b2=b2)
            blocks.append(blk)
        stages.append(blocks)
    stem_b, stem_w = ws[i], ws[i + 1]
    return fc_b, fc_w, head_b, head_w, stages, stem_b, stem_w


def kernel(x, *ws):
    fc_b, fc_w, head_b, head_w, stages, stem_b, stem_w = _unpack(list(ws))
    B = x.shape[0]

    xh = jnp.transpose(x, (0, 2, 3, 1)).astype(jnp.bfloat16)
    xq = jnp.pad(xh, ((0, 0), (3, 3), (3, 3), (0, 0)))
    cols = [xq[:, di:di + 2 * 114:2, dj:dj + 2 * 114:2, :]
            for di in range(3) for dj in range(3)]
    pt = jnp.concatenate(cols, axis=-1)
    cfgs1 = [(3, False, False, False), (3, False, True, False)]
    prms1 = [stages[0][0], stages[0][1]]
    h = _chain_call(
        [pt, stem_w, stem_b],
        [pl.BlockSpec((1, 114, 114, 27), lambda b: (b, 0, 0, 0)),
         _const_spec(stem_w.shape), _const_spec(stem_b.shape)],
        cfgs1, prms1, out_shape=(B, 114, 114, 16), out_pad=1,
        stem_hw=(112, 112),
        flops=2 * 114 * 114 * 27 * 32 + _chain_flops(cfgs1, prms1, 112))

    chain_plan = [
        ([(0, 3)], 56, 2, 1),
        ([(1, 3)], 28, 1, 2),
        ([(2, 4), (3, 4)], 14, 2, 2),
        ([(4, 5)], 7, 1, 4),
    ]
    for chain, hw_out, out_pad, G in chain_plan:
        packed = _parity_pack(h)
        cfgs, prms = [], []
        first = True
        for sidx, nlayers in chain:
            expand, k, stride, cin, cout, layers = _STAGES[sidx + 1]
            for li in range(nlayers):
                is_s2 = first and li == 0 and stride == 2
                skip = li > 0
                cfgs.append((k, True, skip, is_s2))
                prms.append(stages[sidx + 1][li])
            first = False
        H_in = (h.shape[1] - 2 * ((cfgs[0][0] - 1) // 2))
        hwp = hw_out + 2 * out_pad
        h = _chain_call(
            [packed],
            [pl.BlockSpec((G,) + packed.shape[1:],
                          lambda b: (b, 0, 0, 0, 0))],
            cfgs, prms,
            from_planes=True, s2_hw=(H_in, H_in),
            out_shape=(B, hwp, hwp, prms[-1]["pj_w"].shape[1]),
            out_pad=out_pad, G=G,
            flops=_chain_flops(cfgs, prms, H_in))

    cfgs6 = [(3, True, False, False), (3, True, True, False)]
    prms6 = [stages[6][0], stages[6][1]]
    logits, feat = _chain_call(
        [h],
        [pl.BlockSpec((4,) + h.shape[1:], lambda b: (b, 0, 0, 0))],
        cfgs6, prms6, head=(head_w, head_b, fc_w, fc_b), G=4,
        flops=_chain_flops(cfgs6, prms6, 7) + 2 * 49 * 352 * 1408,
    )
    return logits.reshape(B, 10), feat.reshape(B, 1408)


# speedup vs baseline: 1.0912x; 1.0912x over previous
import math
from functools import partial

import jax
import jax.numpy as jnp
from jax.experimental import pallas as pl
from jax.experimental.pallas import tpu as pltpu

_MIB = 1024 * 1024

_STAGES = (
    (1, 3, 1, 32, 16, 2),
    (6, 3, 2, 16, 24, 3),
    (6, 5, 2, 24, 48, 3),
    (6, 3, 2, 48, 88, 4),
    (6, 5, 1, 88, 120, 4),
    (6, 5, 2, 120, 208, 5),
    (6, 3, 1, 208, 352, 2),
)

_PKEYS = ("ex_w", "ex_b", "dw_w", "dw_b", "se_w1", "se_b1",
          "se_w2", "se_b2", "pj_w", "pj_b")
_PKEYS_NOEX = _PKEYS[2:]


def _silu(v):
    return v * jax.nn.sigmoid(v)


def _const_spec(shape):
    n = len(shape)
    return pl.BlockSpec(shape, lambda b: (0,) * n)


def _pad_hw(v, pad):
    if pad == 0:
        return v
    H, W, C = v.shape
    zc = jnp.zeros((H, pad, C), v.dtype)
    v = jnp.concatenate([zc, v, zc], axis=1)
    zr = jnp.zeros((pad, W + 2 * pad, C), v.dtype)
    return jnp.concatenate([zr, v, zr], axis=0)


def _se_gate(pool, w1, b1, w2, b2):
    h = jnp.dot(pool, w1[...], preferred_element_type=jnp.float32) + b1[...]
    h = _silu(h)
    g = jnp.dot(h, w2[...], preferred_element_type=jnp.float32) + b2[...]
    return jax.nn.sigmoid(g)


def _halo_mask(hq, Hp, Wp, pad, H, W):
    ri = jax.lax.broadcasted_iota(jnp.int32, (Hp, Wp, 1), 0)
    ci = jax.lax.broadcasted_iota(jnp.int32, (Hp, Wp, 1), 1)
    inside = ((ri >= pad) & (ri < pad + H) & (ci >= pad) & (ci < pad + W))
    return jnp.where(inside, hq, 0.0)


def _project(y, acc_skip, blk, Ho, Wo, Cexp):
    (w1, b1, w2, b2, wp_ref, bp_ref) = blk
    pool = jnp.mean(y.reshape(Ho * Wo, Cexp), axis=0, keepdims=True)
    g = _se_gate(pool, w1, b1, w2, b2)
    yb = y.astype(jnp.bfloat16).reshape(Ho * Wo, Cexp) * g.astype(jnp.bfloat16)
    out = jnp.dot(yb, wp_ref[...], preferred_element_type=jnp.float32)
    out = out + bp_ref[...]
    if acc_skip is not None:
        out = out + acc_skip.astype(jnp.float32)
    Cout = out.shape[1]
    return out.astype(jnp.bfloat16).reshape(Ho, Wo, Cout)


def _run_s1_block(xpv, blk, k, expand, skip):
    pad = (k - 1) // 2
    Hp, Wp, Cin = xpv.shape
    H, W = Hp - 2 * pad, Wp - 2 * pad
    if expand:
        we_ref, be_ref = blk[0], blk[1]
        blk = blk[2:]
        Cexp = we_ref.shape[1]
        hq = jnp.dot(xpv.reshape(Hp * Wp, Cin), we_ref[...],
                     preferred_element_type=jnp.float32) + be_ref[...]
        hq = _silu(hq).reshape(Hp, Wp, Cexp)
        hb = _halo_mask(hq, Hp, Wp, pad, H, W).astype(jnp.bfloat16)
    else:
        hb = xpv
        Cexp = Cin
    wd_ref, bd_ref = blk[0], blk[1]
    acc = None
    for di in range(k):
        band = hb[di:di + H, :, :].astype(jnp.float32)
        for dj in range(k):
            t = di * k + dj
            wt = wd_ref[t:t + 1, :][None]
            c = band[:, dj:dj + W, :] * wt
            acc = c if acc is None else acc + c
    y = _silu(acc + bd_ref[...][None])
    sk = None
    if skip:
        sk = xpv[pad:pad + H, pad:pad + W, :].reshape(H * W, Cin)
    return _project(y, sk, blk[2:], H, W, Cexp)


def _run_s2_block(planes_in, blk, k, H, W):
    pad = (k - 1) // 2
    Hp, Wp = H + 2 * pad, W + 2 * pad
    Ho = (Hp - k) // 2 + 1
    Wo = (Wp - k) // 2 + 1
    we_ref, be_ref, wd_ref, bd_ref = blk[0], blk[1], blk[2], blk[3]
    Cexp = we_ref.shape[1]
    planes = []
    for r in range(2):
        for s in range(2):
            pv = planes_in[2 * r + s]
            hs, ws, Cin = pv.shape
            hq = jnp.dot(pv.reshape(hs * ws, Cin), we_ref[...],
                         preferred_element_type=jnp.float32) + be_ref[...]
            hq = _silu(hq).reshape(hs, ws, Cexp)
            ri = jax.lax.broadcasted_iota(jnp.int32, (hs, ws, 1), 0)
            ci = jax.lax.broadcasted_iota(jnp.int32, (hs, ws, 1), 1)
            inside = ((r + 2 * ri >= pad) & (r + 2 * ri < pad + H)
                      & (s + 2 * ci >= pad) & (s + 2 * ci < pad + W))
            hq = jnp.where(inside, hq, 0.0)
            planes.append(hq.astype(jnp.bfloat16))
    acc = None
    for r in range(2):
        for s in range(2):
            xr = planes[2 * r + s]
            na = (k - 1 - r) // 2 + 1
            nb = (k - 1 - s) // 2 + 1
            for a in range(na):
                band = xr[a:a + Ho, :, :].astype(jnp.float32)
                for b2_ in range(nb):
                    di, dj = 2 * a + r, 2 * b2_ + s
                    t = di * k + dj
                    wt = wd_ref[t:t + 1, :][None]
                    c = band[:, b2_:b2_ + Wo, :] * wt
                    acc = c if acc is None else acc + c
    y = _silu(acc + bd_ref[...][None])
    return _project(y, None, blk[4:], Ho, Wo, Cexp)


def _chain_body(*refs, cfgs, from_planes, s2_hw, out_pad, has_head, stem_hw, G):
    for g in range(G):
        i = 0
        planes = None
        xv = None
        if stem_hw is not None:
            H, W = stem_hw
            Hp, Wp = H + 2, W + 2
            pt_ref, ws_ref, bs_ref = refs[0], refs[1], refs[2]
            i = 3
            Cstem = ws_ref.shape[1]
            h = jnp.dot(pt_ref[g].reshape(Hp * Wp, 27), ws_ref[...],
                        preferred_element_type=jnp.float32) + bs_ref[...]
            h = _silu(h).reshape(Hp, Wp, Cstem)
            xv = _halo_mask(h, Hp, Wp, 1, H, W).astype(jnp.bfloat16)
        elif from_planes:
            v = refs[0][g]
            Cin = v.shape[3] // 2
            planes = [v[:, r, :, s * Cin:(s + 1) * Cin]
                      for r in range(2) for s in range(2)]
            i = 1
        else:
            xv = refs[0][g]
            i = 1

        out = None
        for bi, (k, expand, skip, is_s2) in enumerate(cfgs):
            n = 10 if expand else 8
            blk = refs[i:i + n]
            i += n
            if is_s2:
                out = _run_s2_block(planes, blk, k, s2_hw[0], s2_hw[1])
            else:
                out = _run_s1_block(xv, blk, k, expand, skip)
            if bi + 1 < len(cfgs):
                xv = _pad_hw(out, (cfgs[bi + 1][0] - 1) // 2)

        if has_head:
            wh_ref, bh_ref, wf_ref, bf_ref = refs[i:i + 4]
            lo_ref, fe_ref = refs[i + 4], refs[i + 5]
            Ho, Wo, C = out.shape
            hh = jnp.dot(out.reshape(Ho * Wo, C), wh_ref[...],
                         preferred_element_type=jnp.float32) + bh_ref[...]
            hh = _silu(hh)
            hb = hh.astype(jnp.bfloat16).astype(jnp.float32)
            feat = jnp.mean(hb, axis=0, keepdims=True)
            lo = jnp.dot(feat, wf_ref[...], preferred_element_type=jnp.float32)
            lo_ref[g] = lo + bf_ref[...]
            fe_ref[g] = feat
        else:
            refs[i][g] = _pad_hw(out, out_pad)


def _params(step_bytes):
    return pltpu.CompilerParams(
        dimension_semantics=("parallel",),
        vmem_limit_bytes=int(min(48 * _MIB, max(32 * _MIB, 3 * int(step_bytes)))))


def _blk_refs(prm, expand):
    keys = _PKEYS if expand else _PKEYS_NOEX
    return [prm[key] for key in keys]


def _chain_call(inputs, in_specs, cfgs, prms, *, from_planes=False, s2_hw=None,
                out_shape=None, out_pad=0, head=None, stem_hw=None, flops=0,
                G=1):
    args = list(inputs)
    specs = list(in_specs)
    for cfg, prm in zip(cfgs, prms):
        for a in _blk_refs(prm, cfg[1]):
            args.append(a)
            specs.append(_const_spec(a.shape))
    B = inputs[0].shape[0]
    if head is not None:
        head_w, head_b, fc_w, fc_b = head
        for a in (head_w, head_b, fc_w, fc_b):
            args.append(a)
            specs.append(_const_spec(a.shape))
        N, ncls = head_w.shape[1], fc_w.shape[1]
        out_specs = [pl.BlockSpec((G, 1, ncls), lambda b: (b, 0, 0)),
                     pl.BlockSpec((G, 1, N), lambda b: (b, 0, 0))]
        out_sds = (jax.ShapeDtypeStruct((B, 1, ncls), jnp.float32),
                   jax.ShapeDtypeStruct((B, 1, N), jnp.float32))
    else:
        out_specs = pl.BlockSpec((G,) + out_shape[1:],
                                 lambda b: (b, 0, 0, 0))
        out_sds = jax.ShapeDtypeStruct(out_shape, jnp.bfloat16)
    wbytes = sum(int(a.size) * a.dtype.itemsize for a in args[len(inputs):])
    in_bytes = sum(int(a.size) * a.dtype.itemsize for a in inputs) // B
    step = 4 * G * in_bytes + wbytes + 8 * _MIB
    return pl.pallas_call(
        partial(_chain_body, cfgs=tuple(cfgs), from_planes=from_planes,
                s2_hw=s2_hw, out_pad=out_pad, has_head=head is not None,
                stem_hw=stem_hw, G=G),
        grid=(B // G,),
        in_specs=specs,
        out_specs=out_specs,
        out_shape=out_sds,
        compiler_params=_params(step),
        cost_estimate=pl.CostEstimate(
            flops=int(flops), transcendentals=int(flops // 16),
            bytes_accessed=int(B * in_bytes + wbytes)),
    )(*args)


def _parity_pack(x):
    B, Hp, Wp, C = x.shape
    return x.reshape(B, Hp // 2, 2, Wp // 2, 2 * C)


def _chain_flops(cfgs, prms, hw_in):
    f = 0
    h = hw_in
    for (k, expand, _skip, is_s2), prm in zip(cfgs, prms):
        cexp = prm["dw_w"].shape[1]
        cout = prm["pj_w"].shape[1]
        ho = h // 2 if is_s2 else h
        cin = prm["ex_w"].shape[0] if expand else cexp
        f += 2 * (h * h * cin * cexp * int(expand)
                  + k * k * ho * ho * cexp + ho * ho * cexp * cout)
        h = ho
    return f


def _unpack(ws):
    fc_b, fc_w, head_b, head_w = ws[0], ws[1], ws[2], ws[3]
    i = 4
    stages = []
    for (expand, _k, _s, _ci, _co, layers) in _STAGES:
        blocks = []
        for _li in range(layers):
            if expand == 1:
                dw_b, dw_w, pj_b, pj_w, b1, b2, w1, w2 = ws[i:i + 8]
                i += 8
                blk = dict(dw_w=dw_w, dw_b=dw_b, pj_w=pj_w, pj_b=pj_b,
                           se_w1=w1, se_b1=b1, se_w2=w2, se_b2=b2)
            else:
                dw_b, dw_w, ex_b, ex_w, pj_b, pj_w, b1, b2, w1, w2 = ws[i:i + 10]
                i += 10
                blk = dict(dw_w=dw_w, dw_b=dw_b, ex_w=ex_w, ex_b=ex_b,
                           pj_w=pj_w, pj_b=pj_b,
                           se_w1=w1, se_b1=b1, se_w2=w2, se_b2=b2)
            blocks.append(blk)
        stages.append(blocks)
    stem_b, stem_w = ws[i], ws[i + 1]
    return fc_b, fc_w, head_b, head_w, stages, stem_b, stem_w


def kernel(x, *ws):
    fc_b, fc_w, head_b, head_w, stages, stem_b, stem_w = _unpack(list(ws))
    B = x.shape[0]

    xh = jnp.transpose(x, (0, 2, 3, 1)).astype(jnp.bfloat16)
    xq = jnp.pad(xh, ((0, 0), (3, 3), (3, 3), (0, 0)))
    cols = [xq[:, di:di + 2 * 114:2, dj:dj + 2 * 114:2, :]
            for di in range(3) for dj in range(3)]
    pt = jnp.concatenate(cols, axis=-1)
    cfgs1 = [(3, False, False, False), (3, False, True, False)]
    prms1 = [stages[0][0], stages[0][1]]
    h = _chain_call(
        [pt, stem_w, stem_b],
        [pl.BlockSpec((1, 114, 114, 27), lambda b: (b, 0, 0, 0)),
         _const_spec(stem_w.shape), _const_spec(stem_b.shape)],
        cfgs1, prms1, out_shape=(B, 114, 114, 16), out_pad=1,
        stem_hw=(112, 112),
        flops=2 * 114 * 114 * 27 * 32 + _chain_flops(cfgs1, prms1, 112))

    chain_plan = [
        ([(0, 3)], 56, 2, 1),
        ([(1, 3)], 28, 1, 1),
        ([(2, 4), (3, 4)], 14, 2, 1),
        ([(4, 5)], 7, 1, 1),
    ]
    for chain, hw_out, out_pad, G in chain_plan:
        packed = _parity_pack(h)
        cfgs, prms = [], []
        first = True
        for sidx, nlayers in chain:
            expand, k, stride, cin, cout, layers = _STAGES[sidx + 1]
            for li in range(nlayers):
                is_s2 = first and li == 0 and stride == 2
                skip = li > 0
                cfgs.append((k, True, skip, is_s2))
                prms.append(stages[sidx + 1][li])
            first = False
        H_in = (h.shape[1] - 2 * ((cfgs[0][0] - 1) // 2))
        hwp = hw_out + 2 * out_pad
        h = _chain_call(
            [packed],
            [pl.BlockSpec((G,) + packed.shape[1:],
                          lambda b: (b, 0, 0, 0, 0))],
            cfgs, prms,
            from_planes=True, s2_hw=(H_in, H_in),
            out_shape=(B, hwp, hwp, prms[-1]["pj_w"].shape[1]),
            out_pad=out_pad, G=G,
            flops=_chain_flops(cfgs, prms, H_in))

    cfgs6 = [(3, True, False, False), (3, True, True, False)]
    prms6 = [stages[6][0], stages[6][1]]
    logits, feat = _chain_call(
        [h],
        [pl.BlockSpec((1,) + h.shape[1:], lambda b: (b, 0, 0, 0))],
        cfgs6, prms6, head=(head_w, head_b, fc_w, fc_b), G=1,
        flops=_chain_flops(cfgs6, prms6, 7) + 2 * 49 * 352 * 1408,
    )
    return logits.reshape(B, 10), feat.reshape(B, 1408)


# speedup vs baseline: 1.1816x; 1.0828x over previous
import math
from functools import partial

import jax
import jax.numpy as jnp
from jax.experimental import pallas as pl
from jax.experimental.pallas import tpu as pltpu

_MIB = 1024 * 1024

_STAGES = (
    (1, 3, 1, 32, 16, 2),
    (6, 3, 2, 16, 24, 3),
    (6, 5, 2, 24, 48, 3),
    (6, 3, 2, 48, 88, 4),
    (6, 5, 1, 88, 120, 4),
    (6, 5, 2, 120, 208, 5),
    (6, 3, 1, 208, 352, 2),
)

_PKEYS = ("ex_w", "ex_b", "dw_w", "dw_b", "se_w1", "se_b1",
          "se_w2", "se_b2", "pj_w", "pj_b")
_PKEYS_NOEX = _PKEYS[2:]


def _silu(v):
    return v * jax.nn.sigmoid(v)


def _const_spec(shape):
    n = len(shape)
    return pl.BlockSpec(shape, lambda b: (0,) * n)


def _pad_hw(v, pad):
    if pad == 0:
        return v
    H, W, C = v.shape
    zc = jnp.zeros((H, pad, C), v.dtype)
    v = jnp.concatenate([zc, v, zc], axis=1)
    zr = jnp.zeros((pad, W + 2 * pad, C), v.dtype)
    return jnp.concatenate([zr, v, zr], axis=0)


def _se_gate(pool, w1, b1, w2, b2):
    h = jnp.dot(pool, w1[...], preferred_element_type=jnp.float32) + b1[...]
    h = _silu(h)
    g = jnp.dot(h, w2[...], preferred_element_type=jnp.float32) + b2[...]
    return jax.nn.sigmoid(g)


def _halo_mask(hq, Hp, Wp, pad, H, W):
    ri = jax.lax.broadcasted_iota(jnp.int32, (Hp, Wp, 1), 0)
    ci = jax.lax.broadcasted_iota(jnp.int32, (Hp, Wp, 1), 1)
    inside = ((ri >= pad) & (ri < pad + H) & (ci >= pad) & (ci < pad + W))
    return jnp.where(inside, hq, 0.0)


def _project(y, acc_skip, blk, Ho, Wo, Cexp):
    (w1, b1, w2, b2, wp_ref, bp_ref) = blk
    pool = jnp.mean(y.reshape(Ho * Wo, Cexp), axis=0, keepdims=True)
    g = _se_gate(pool, w1, b1, w2, b2)
    yb = y.astype(jnp.bfloat16).reshape(Ho * Wo, Cexp) * g.astype(jnp.bfloat16)
    out = jnp.dot(yb, wp_ref[...], preferred_element_type=jnp.float32)
    out = out + bp_ref[...]
    if acc_skip is not None:
        out = out + acc_skip.astype(jnp.float32)
    Cout = out.shape[1]
    return out.astype(jnp.bfloat16).reshape(Ho, Wo, Cout)


def _run_s1_block(xpv, blk, k, expand, skip):
    pad = (k - 1) // 2
    Hp, Wp, Cin = xpv.shape
    H, W = Hp - 2 * pad, Wp - 2 * pad
    if expand:
        we_ref, be_ref = blk[0], blk[1]
        blk = blk[2:]
        Cexp = we_ref.shape[1]
        hq = jnp.dot(xpv.reshape(Hp * Wp, Cin), we_ref[...],
                     preferred_element_type=jnp.float32) + be_ref[...]
        hq = _silu(hq).reshape(Hp, Wp, Cexp)
        hb = _halo_mask(hq, Hp, Wp, pad, H, W).astype(jnp.bfloat16)
    else:
        hb = xpv
        Cexp = Cin
    wd_ref, bd_ref = blk[0], blk[1]
    acc = None
    for di in range(k):
        band = hb[di:di + H, :, :].astype(jnp.float32)
        for dj in range(k):
            t = di * k + dj
            wt = wd_ref[t:t + 1, :][None]
            c = band[:, dj:dj + W, :] * wt
            acc = c if acc is None else acc + c
    y = _silu(acc + bd_ref[...][None])
    sk = None
    if skip:
        sk = xpv[pad:pad + H, pad:pad + W, :].reshape(H * W, Cin)
    return _project(y, sk, blk[2:], H, W, Cexp)


def _run_s2_block(planes_in, blk, k, H, W):
    pad = (k - 1) // 2
    Hp, Wp = H + 2 * pad, W + 2 * pad
    Ho = (Hp - k) // 2 + 1
    Wo = (Wp - k) // 2 + 1
    we_ref, be_ref, wd_ref, bd_ref = blk[0], blk[1], blk[2], blk[3]
    Cexp = we_ref.shape[1]
    planes = []
    for r in range(2):
        for s in range(2):
            pv = planes_in[2 * r + s]
            hs, ws, Cin = pv.shape
            hq = jnp.dot(pv.reshape(hs * ws, Cin), we_ref[...],
                         preferred_element_type=jnp.float32) + be_ref[...]
            hq = _silu(hq).reshape(hs, ws, Cexp)
            ri = jax.lax.broadcasted_iota(jnp.int32, (hs, ws, 1), 0)
            ci = jax.lax.broadcasted_iota(jnp.int32, (hs, ws, 1), 1)
            inside = ((r + 2 * ri >= pad) & (r + 2 * ri < pad + H)
                      & (s + 2 * ci >= pad) & (s + 2 * ci < pad + W))
            hq = jnp.where(inside, hq, 0.0)
            planes.append(hq.astype(jnp.bfloat16))
    acc = None
    for r in range(2):
        for s in range(2):
            xr = planes[2 * r + s]
            na = (k - 1 - r) // 2 + 1
            nb = (k - 1 - s) // 2 + 1
            for a in range(na):
                band = xr[a:a + Ho, :, :].astype(jnp.float32)
                for b2_ in range(nb):
                    di, dj = 2 * a + r, 2 * b2_ + s
                    t = di * k + dj
                    wt = wd_ref[t:t + 1, :][None]
                    c = band[:, b2_:b2_ + Wo, :] * wt
                    acc = c if acc is None else acc + c
    y = _silu(acc + bd_ref[...][None])
    return _project(y, None, blk[4:], Ho, Wo, Cexp)


def _chain_body(*refs, cfgs, from_planes, s2_hw, out_pad, has_head, stem_hw, G):
    for g in range(G):
        i = 0
        planes = None
        xv = None
        if stem_hw is not None:
            H, W = stem_hw
            Hp, Wp = H + 2, W + 2
            pt_ref, ws_ref, bs_ref = refs[0], refs[1], refs[2]
            i = 3
            Cstem = ws_ref.shape[1]
            h = jnp.dot(pt_ref[g].reshape(Hp * Wp, 27), ws_ref[...],
                        preferred_element_type=jnp.float32) + bs_ref[...]
            h = _silu(h).reshape(Hp, Wp, Cstem)
            xv = _halo_mask(h, Hp, Wp, 1, H, W).astype(jnp.bfloat16)
        elif from_planes:
            v = refs[0][g]
            Cin = v.shape[3] // 2
            planes = [v[:, r, :, s * Cin:(s + 1) * Cin]
                      for r in range(2) for s in range(2)]
            i = 1
        else:
            xv = refs[0][g]
            i = 1

        out = None
        for bi, (k, expand, skip, is_s2) in enumerate(cfgs):
            n = 10 if expand else 8
            blk = refs[i:i + n]
            i += n
            if is_s2:
                out = _run_s2_block(planes, blk, k, s2_hw[0], s2_hw[1])
            else:
                out = _run_s1_block(xv, blk, k, expand, skip)
            if bi + 1 < len(cfgs):
                xv = _pad_hw(out, (cfgs[bi + 1][0] - 1) // 2)

        if has_head:
            wh_ref, bh_ref, wf_ref, bf_ref = refs[i:i + 4]
            lo_ref, fe_ref = refs[i + 4], refs[i + 5]
            Ho, Wo, C = out.shape
            hh = jnp.dot(out.reshape(Ho * Wo, C), wh_ref[...],
                         preferred_element_type=jnp.float32) + bh_ref[...]
            hh = _silu(hh)
            hb = hh.astype(jnp.bfloat16).astype(jnp.float32)
            feat = jnp.mean(hb, axis=0, keepdims=True)
            lo = jnp.dot(feat, wf_ref[...], preferred_element_type=jnp.float32)
            lo_ref[g] = lo + bf_ref[...]
            fe_ref[g] = feat
        else:
            refs[i][g] = _pad_hw(out, out_pad)


def _params(step_bytes):
    return pltpu.CompilerParams(
        dimension_semantics=("parallel",),
        vmem_limit_bytes=int(min(48 * _MIB, max(32 * _MIB, 3 * int(step_bytes)))))


def _blk_refs(prm, expand):
    keys = _PKEYS if expand else _PKEYS_NOEX
    return [prm[key] for key in keys]


def _chain_call(inputs, in_specs, cfgs, prms, *, from_planes=False, s2_hw=None,
                out_shape=None, out_pad=0, head=None, stem_hw=None, flops=0,
                G=1):
    args = list(inputs)
    specs = list(in_specs)
    for cfg, prm in zip(cfgs, prms):
        for a in _blk_refs(prm, cfg[1]):
            args.append(a)
            specs.append(_const_spec(a.shape))
    B = inputs[0].shape[0]
    if head is not None:
        head_w, head_b, fc_w, fc_b = head
        for a in (head_w, head_b, fc_w, fc_b):
            args.append(a)
            specs.append(_const_spec(a.shape))
        N, ncls = head_w.shape[1], fc_w.shape[1]
        out_specs = [pl.BlockSpec((G, 1, ncls), lambda b: (b, 0, 0)),
                     pl.BlockSpec((G, 1, N), lambda b: (b, 0, 0))]
        out_sds = (jax.ShapeDtypeStruct((B, 1, ncls), jnp.float32),
                   jax.ShapeDtypeStruct((B, 1, N), jnp.float32))
    else:
        out_specs = pl.BlockSpec((G,) + out_shape[1:],
                                 lambda b: (b, 0, 0, 0))
        out_sds = jax.ShapeDtypeStruct(out_shape, jnp.bfloat16)
    wbytes = sum(int(a.size) * a.dtype.itemsize for a in args[len(inputs):])
    in_bytes = sum(int(a.size) * a.dtype.itemsize for a in inputs) // B
    step = 4 * G * in_bytes + wbytes + 8 * _MIB
    return pl.pallas_call(
        partial(_chain_body, cfgs=tuple(cfgs), from_planes=from_planes,
                s2_hw=s2_hw, out_pad=out_pad, has_head=head is not None,
                stem_hw=stem_hw, G=G),
        grid=(B // G,),
        in_specs=specs,
        out_specs=out_specs,
        out_shape=out_sds,
        compiler_params=_params(step),
        cost_estimate=pl.CostEstimate(
            flops=int(flops), transcendentals=int(flops // 16),
            bytes_accessed=int(B * in_bytes + wbytes)),
    )(*args)


def _parity_pack(x):
    B, Hp, Wp, C = x.shape
    return x.reshape(B, Hp // 2, 2, Wp // 2, 2 * C)


def _chain_flops(cfgs, prms, hw_in):
    f = 0
    h = hw_in
    for (k, expand, _skip, is_s2), prm in zip(cfgs, prms):
        cexp = prm["dw_w"].shape[1]
        cout = prm["pj_w"].shape[1]
        ho = h // 2 if is_s2 else h
        cin = prm["ex_w"].shape[0] if expand else cexp
        f += 2 * (h * h * cin * cexp * int(expand)
                  + k * k * ho * ho * cexp + ho * ho * cexp * cout)
        h = ho
    return f


def _unpack(ws):
    fc_b, fc_w, head_b, head_w = ws[0], ws[1], ws[2], ws[3]
    i = 4
    stages = []
    for (expand, _k, _s, _ci, _co, layers) in _STAGES:
        blocks = []
        for _li in range(layers):
            if expand == 1:
                dw_b, dw_w, pj_b, pj_w, b1, b2, w1, w2 = ws[i:i + 8]
                i += 8
                blk = dict(dw_w=dw_w, dw_b=dw_b, pj_w=pj_w, pj_b=pj_b,
                           se_w1=w1, se_b1=b1, se_w2=w2, se_b2=b2)
            else:
                dw_b, dw_w, ex_b, ex_w, pj_b, pj_w, b1, b2, w1, w2 = ws[i:i + 10]
                i += 10
                blk = dict(dw_w=dw_w, dw_b=dw_b, ex_w=ex_w, ex_b=ex_b,
                           pj_w=pj_w, pj_b=pj_b,
                           se_w1=w1, se_b1=b1, se_w2=w2, se_b2=b2)
            blocks.append(blk)
        stages.append(blocks)
    stem_b, stem_w = ws[i], ws[i + 1]
    return fc_b, fc_w, head_b, head_w, stages, stem_b, stem_w


def kernel(x, *ws):
    fc_b, fc_w, head_b, head_w, stages, stem_b, stem_w = _unpack(list(ws))
    B = x.shape[0]

    xh = jnp.transpose(x, (0, 2, 3, 1)).astype(jnp.bfloat16)
    xq = jnp.pad(xh, ((0, 0), (3, 3), (3, 3), (0, 0)))
    cols = [xq[:, di:di + 2 * 114:2, dj:dj + 2 * 114:2, :]
            for di in range(3) for dj in range(3)]
    pt = jnp.concatenate(cols, axis=-1)
    cfgs1 = [(3, False, False, False), (3, False, True, False)]
    prms1 = [stages[0][0], stages[0][1]]
    h = _chain_call(
        [pt, stem_w, stem_b],
        [pl.BlockSpec((1, 114, 114, 27), lambda b: (b, 0, 0, 0)),
         _const_spec(stem_w.shape), _const_spec(stem_b.shape)],
        cfgs1, prms1, out_shape=(B, 114, 114, 16), out_pad=1,
        stem_hw=(112, 112),
        flops=2 * 114 * 114 * 27 * 32 + _chain_flops(cfgs1, prms1, 112))

    hw = 112
    for sidx in range(1, 6):
        _ex, k, stride, _ci, cout, layers = _STAGES[sidx]
        for li in range(layers):
            is_s2 = (li == 0 and stride == 2)
            if li + 1 < layers:
                k_next = k
            else:
                k_next = _STAGES[sidx + 1][1]
            out_pad = (k_next - 1) // 2
            prm = stages[sidx][li]
            cfg = (k, True, li > 0, is_s2)
            hw2 = hw // 2 if is_s2 else hw
            hwp = hw2 + 2 * out_pad
            if is_s2:
                packed = _parity_pack(h)
                inputs = [packed]
                specs = [pl.BlockSpec((1,) + packed.shape[1:],
                                      lambda b: (b, 0, 0, 0, 0))]
            else:
                inputs = [h]
                specs = [pl.BlockSpec((1,) + h.shape[1:],
                                      lambda b: (b, 0, 0, 0))]
            h = _chain_call(
                inputs, specs, [cfg], [prm],
                from_planes=is_s2, s2_hw=(hw, hw),
                out_shape=(B, hwp, hwp, cout),
                out_pad=out_pad,
                flops=_chain_flops([cfg], [prm], hw))
            hw = hw2

    cfgs6 = [(3, True, False, False), (3, True, True, False)]
    prms6 = [stages[6][0], stages[6][1]]
    logits, feat = _chain_call(
        [h],
        [pl.BlockSpec((1,) + h.shape[1:], lambda b: (b, 0, 0, 0))],
        cfgs6, prms6, head=(head_w, head_b, fc_w, fc_b), G=1,
        flops=_chain_flops(cfgs6, prms6, 7) + 2 * 49 * 352 * 1408,
    )
    return logits.reshape(B, 10), feat.reshape(B, 1408)
